# baseline (speedup 1.0000x reference)
import functools

import jax
import jax.numpy as jnp
from jax import lax
from jax.experimental import pallas as pl
from jax.experimental.pallas import tpu as pltpu
from jax.experimental.pallas import tpu_sc as plsc

_LANES = 128
_L = 16


def _energy_body(x_ref, e_ref):
    e_ref[...] = jnp.sum(x_ref[...], axis=1)


def _sc_threshold_body(N, k, n_shards, e_hbm, t_hbm, ebuf, kbuf, hist,
                       l256, tmp256, g256, tbuf, sh_hist):
    c = lax.axis_index("c")
    s = lax.axis_index("s")
    groups_per_core = 16 // n_shards
    b = c * groups_per_core + s // n_shards
    shard = s % n_shards
    shard_n = N // n_shards
    n_vecs = shard_n // _L

    lanes = lax.iota(jnp.int32, _L)
    ones = jnp.ones((_L,), jnp.int32)

    pltpu.sync_copy(e_hbm.at[b, pl.ds(shard * shard_n, shard_n)], ebuf)

    def prekey(i, _):
        for j in range(8):
            off = i * (8 * _L) + j * _L
            u = plsc.bitcast(ebuf[pl.ds(off, _L)], jnp.uint32)
            key = jnp.where(u >= jnp.uint32(0x80000000), ~u,
                            u | jnp.uint32(0x80000000))
            kbuf[pl.ds(off, _L)] = key
        return 0

    lax.fori_loop(0, n_vecs // 8, prekey, 0)

    t_part = jnp.uint32(0)
    r_k = jnp.int32(k)

    for p in range(4):
        shift = 24 - 8 * p

        def clr(i, _):
            for j in range(16):
                hist[i, pl.ds(j * _L, _L)] = jnp.zeros((_L,), jnp.int32)
            return 0

        lax.fori_loop(0, 16, clr, 0)

        def count(i, _):
            for j in range(8):
                off = i * (8 * _L) + j * _L
                key = kbuf[pl.ds(off, _L)]
                dig = (jnp.right_shift(key, jnp.uint32(shift))
                       & jnp.uint32(0xFF)).astype(jnp.int32)
                if p == 0:
                    plsc.addupdate_scatter(hist, [lanes, dig], ones)
                else:
                    m = jnp.right_shift(key, jnp.uint32(shift + 8)) == \
                        (t_part >> jnp.uint32(shift + 8))
                    plsc.addupdate_scatter(hist, [lanes, dig], ones, mask=m)
            return 0

        lax.fori_loop(0, n_vecs // 8, count, 0)

        def mrg(j, _):
            acc = hist[0, pl.ds(j * _L, _L)]
            for l in range(1, 16):
                acc = acc + hist[l, pl.ds(j * _L, _L)]
            l256[pl.ds(j * _L, _L)] = acc
            return 0

        lax.fori_loop(0, 16, mrg, 0)

        pltpu.sync_copy(l256, sh_hist.at[s])
        plsc.subcore_barrier()
        base_row = (s // n_shards) * n_shards
        for r in range(n_shards):
            pltpu.sync_copy(sh_hist.at[base_row + r], tmp256)

            def accj(j, _, first=(r == 0)):
                cur = tmp256[pl.ds(j * _L, _L)]
                if first:
                    g256[pl.ds(j * _L, _L)] = cur
                else:
                    g256[pl.ds(j * _L, _L)] = g256[pl.ds(j * _L, _L)] + cur
                return 0

            lax.fori_loop(0, 16, accj, 0)
        plsc.subcore_barrier()

        def scan_body(i, carry_in):
            carry, found, d, cgt = carry_in
            v = 15 - i
            hvec = g256[pl.ds(v * _L, _L)]
            ssum = jnp.sum(hvec)
            cum = plsc.cumsum(hvec)
            suff = ssum - cum + hvec
            cond = (carry + suff) >= r_k
            anyc = jnp.max(cond.astype(jnp.int32))
            jstar = jnp.max(jnp.where(cond, lanes, -1))
            sj = jnp.sum(jnp.where(lanes == jstar, suff, 0))
            hj = jnp.sum(jnp.where(lanes == jstar, hvec, 0))
            take = (found == 0) & (anyc == 1)
            d = jnp.where(take, v * _L + jstar, d)
            cgt = jnp.where(take, carry + sj - hj, cgt)
            found = jnp.where(take, 1, found)
            carry = carry + ssum
            return carry, found, d, cgt

        _, _, d, cgt = lax.fori_loop(
            0, 16, scan_body,
            (jnp.int32(0), jnp.int32(0), jnp.int32(0), jnp.int32(0)))
        r_k = r_k - cgt
        t_part = t_part | (d.astype(jnp.uint32) << jnp.uint32(shift))

    tvec = jnp.full((_L,), t_part, dtype=jnp.uint32)
    fvec = jnp.where(tvec >= jnp.uint32(0x80000000),
                     tvec ^ jnp.uint32(0x80000000), ~tvec)

    @pl.when(shard == 0)
    def _():
        tbuf[...] = plsc.bitcast(fvec, jnp.float32)
        pltpu.sync_copy(tbuf, t_hbm.at[b])


def _mask_body(x_ref, e_ref, t_ref, o_ref):
    b = pl.program_id(0)
    rows_b = jax.lax.broadcasted_iota(jnp.int32, t_ref.shape, 0)
    t = jnp.max(jnp.where(rows_b == b, t_ref[...], -jnp.inf))
    keep = e_ref[...] >= t
    o_ref[...] = jnp.where(keep[:, None, :, :], x_ref[...], jnp.float32(0.0))


@jax.jit
def kernel(x):
    B, C, H, W = x.shape
    N = H * W
    k = N // 2
    assert N % _LANES == 0
    rows_total = N // _LANES

    n_chunks = 24 if rows_total % 24 == 0 else 1
    rows = rows_total // n_chunks

    xf = x.reshape(B, C, rows_total, _LANES)

    energy = pl.pallas_call(
        _energy_body,
        grid=(B, n_chunks),
        in_specs=[pl.BlockSpec((1, C, rows, _LANES), lambda b, j: (b, 0, j, 0))],
        out_specs=pl.BlockSpec((1, rows, _LANES), lambda b, j: (b, j, 0)),
        out_shape=jax.ShapeDtypeStruct((B, rows_total, _LANES), jnp.float32),
    )(xf)

    assert 16 % (16 * B // 32) == 0 and B % 2 == 0
    n_shards = 32 // B
    shard_n = N // n_shards
    assert shard_n % (8 * _L) == 0 and (shard_n * n_shards) == N

    mesh = plsc.VectorSubcoreMesh(core_axis_name="c", subcore_axis_name="s")
    thresh = pl.kernel(
        functools.partial(_sc_threshold_body, N, k, n_shards),
        mesh=mesh,
        compiler_params=pltpu.CompilerParams(needs_layout_passes=False),
        out_type=jax.ShapeDtypeStruct((B, _L), jnp.float32),
        scratch_types=[
            pltpu.VMEM((shard_n,), jnp.float32),
            pltpu.VMEM((shard_n,), jnp.uint32),
            pltpu.VMEM((16, 256), jnp.int32),
            pltpu.VMEM((256,), jnp.int32),
            pltpu.VMEM((256,), jnp.int32),
            pltpu.VMEM((256,), jnp.int32),
            pltpu.VMEM((_L,), jnp.float32),
            pltpu.VMEM_SHARED((16, 256), jnp.int32),
        ],
    )(energy.reshape(B, N))

    out = pl.pallas_call(
        _mask_body,
        grid=(B, n_chunks),
        in_specs=[
            pl.BlockSpec((1, C, rows, _LANES), lambda b, j: (b, 0, j, 0)),
            pl.BlockSpec((1, rows, _LANES), lambda b, j: (b, j, 0)),
            pl.BlockSpec((B, _L), lambda b, j: (0, 0)),
        ],
        out_specs=pl.BlockSpec((1, C, rows, _LANES), lambda b, j: (b, 0, j, 0)),
        out_shape=jax.ShapeDtypeStruct((B, C, rows_total, _LANES), jnp.float32),
    )(xf, energy, thresh)

    return out.reshape(B, C, H, W)


# speedup vs baseline: 1.0604x; 1.0604x over previous
import functools

import jax
import jax.numpy as jnp
from jax import lax
from jax.experimental import pallas as pl
from jax.experimental.pallas import tpu as pltpu
from jax.experimental.pallas import tpu_sc as plsc

_LANES = 128
_L = 16


def _energy_body(x_ref, e_ref):
    e_ref[...] = jnp.sum(x_ref[...], axis=1)


def _sc_threshold_body(N, k, n_shards, e_hbm, t_hbm, ebuf, kbuf, hist,
                       l256, tmp256, g256, tbuf, sh_hist):
    c = lax.axis_index("c")
    s = lax.axis_index("s")
    groups_per_core = 16 // n_shards
    b = c * groups_per_core + s // n_shards
    shard = s % n_shards
    shard_n = N // n_shards
    n_vecs = shard_n // _L

    lanes = lax.iota(jnp.int32, _L)
    ones = jnp.ones((_L,), jnp.int32)

    pltpu.sync_copy(e_hbm.at[b, pl.ds(shard * shard_n, shard_n)], ebuf)

    def prekey(i, _):
        for j in range(8):
            off = i * (8 * _L) + j * _L
            u = plsc.bitcast(ebuf[pl.ds(off, _L)], jnp.uint32)
            key = jnp.where(u >= jnp.uint32(0x80000000), ~u,
                            u | jnp.uint32(0x80000000))
            kbuf[pl.ds(off, _L)] = key
        return 0

    lax.fori_loop(0, n_vecs // 8, prekey, 0)

    t_part = jnp.uint32(0)
    r_k = jnp.int32(k)

    for p in range(4):
        shift = 24 - 8 * p

        def clr(i, _):
            for j in range(16):
                hist[i, pl.ds(j * _L, _L)] = jnp.zeros((_L,), jnp.int32)
            return 0

        lax.fori_loop(0, 16, clr, 0)

        def count(i, _):
            for j in range(8):
                off = i * (8 * _L) + j * _L
                key = kbuf[pl.ds(off, _L)]
                dig = (jnp.right_shift(key, jnp.uint32(shift))
                       & jnp.uint32(0xFF)).astype(jnp.int32)
                if p == 0:
                    plsc.addupdate_scatter(hist, [lanes, dig], ones)
                else:
                    m = jnp.right_shift(key, jnp.uint32(shift + 8)) == \
                        (t_part >> jnp.uint32(shift + 8))
                    plsc.addupdate_scatter(hist, [lanes, dig], ones, mask=m)
            return 0

        lax.fori_loop(0, n_vecs // 8, count, 0)

        def mrg(j, _):
            acc = hist[0, pl.ds(j * _L, _L)]
            for l in range(1, 16):
                acc = acc + hist[l, pl.ds(j * _L, _L)]
            l256[pl.ds(j * _L, _L)] = acc
            return 0

        lax.fori_loop(0, 16, mrg, 0)

        pltpu.sync_copy(l256, sh_hist.at[s])
        plsc.subcore_barrier()
        base_row = (s // n_shards) * n_shards
        for r in range(n_shards):
            pltpu.sync_copy(sh_hist.at[base_row + r], tmp256)

            def accj(j, _, first=(r == 0)):
                cur = tmp256[pl.ds(j * _L, _L)]
                if first:
                    g256[pl.ds(j * _L, _L)] = cur
                else:
                    g256[pl.ds(j * _L, _L)] = g256[pl.ds(j * _L, _L)] + cur
                return 0

            lax.fori_loop(0, 16, accj, 0)
        plsc.subcore_barrier()

        def scan_body(i, carry_in):
            carry, found, d, cgt = carry_in
            v = 15 - i
            hvec = g256[pl.ds(v * _L, _L)]
            ssum = jnp.sum(hvec)
            cum = plsc.cumsum(hvec)
            suff = ssum - cum + hvec
            cond = (carry + suff) >= r_k
            anyc = jnp.max(cond.astype(jnp.int32))
            jstar = jnp.max(jnp.where(cond, lanes, -1))
            sj = jnp.sum(jnp.where(lanes == jstar, suff, 0))
            hj = jnp.sum(jnp.where(lanes == jstar, hvec, 0))
            take = (found == 0) & (anyc == 1)
            d = jnp.where(take, v * _L + jstar, d)
            cgt = jnp.where(take, carry + sj - hj, cgt)
            found = jnp.where(take, 1, found)
            carry = carry + ssum
            return carry, found, d, cgt

        _, _, d, cgt = lax.fori_loop(
            0, 16, scan_body,
            (jnp.int32(0), jnp.int32(0), jnp.int32(0), jnp.int32(0)))
        r_k = r_k - cgt
        t_part = t_part | (d.astype(jnp.uint32) << jnp.uint32(shift))

    tvec = jnp.full((_L,), t_part, dtype=jnp.uint32)
    fvec = jnp.where(tvec >= jnp.uint32(0x80000000),
                     tvec ^ jnp.uint32(0x80000000), ~tvec)

    @pl.when(shard == 0)
    def _():
        tbuf[...] = plsc.bitcast(fvec, jnp.float32)
        pltpu.sync_copy(tbuf, t_hbm.at[b])


def _mask_body(x_ref, e_ref, t_ref, o_ref):
    b = pl.program_id(0)
    rows_b = jax.lax.broadcasted_iota(jnp.int32, t_ref.shape, 0)
    t = jnp.max(jnp.where(rows_b == b, t_ref[...], -jnp.inf))
    keep = e_ref[...] >= t
    o_ref[...] = jnp.where(keep[:, None, :, :], x_ref[...], jnp.float32(0.0))


@jax.jit
def kernel(x):
    B, C, H, W = x.shape
    N = H * W
    k = N // 2
    assert N % _LANES == 0
    rows_total = N // _LANES

    n_chunks = 8 if rows_total % 8 == 0 else 1
    rows = rows_total // n_chunks

    xf = x.reshape(B, C, rows_total, _LANES)

    energy = pl.pallas_call(
        _energy_body,
        grid=(B, n_chunks),
        in_specs=[pl.BlockSpec((1, C, rows, _LANES), lambda b, j: (b, 0, j, 0))],
        out_specs=pl.BlockSpec((1, rows, _LANES), lambda b, j: (b, j, 0)),
        out_shape=jax.ShapeDtypeStruct((B, rows_total, _LANES), jnp.float32),
    )(xf)

    assert 16 % (16 * B // 32) == 0 and B % 2 == 0
    n_shards = 32 // B
    shard_n = N // n_shards
    assert shard_n % (8 * _L) == 0 and (shard_n * n_shards) == N

    mesh = plsc.VectorSubcoreMesh(core_axis_name="c", subcore_axis_name="s")
    thresh = pl.kernel(
        functools.partial(_sc_threshold_body, N, k, n_shards),
        mesh=mesh,
        compiler_params=pltpu.CompilerParams(needs_layout_passes=False),
        out_type=jax.ShapeDtypeStruct((B, _L), jnp.float32),
        scratch_types=[
            pltpu.VMEM((shard_n,), jnp.float32),
            pltpu.VMEM((shard_n,), jnp.uint32),
            pltpu.VMEM((16, 256), jnp.int32),
            pltpu.VMEM((256,), jnp.int32),
            pltpu.VMEM((256,), jnp.int32),
            pltpu.VMEM((256,), jnp.int32),
            pltpu.VMEM((_L,), jnp.float32),
            pltpu.VMEM_SHARED((16, 256), jnp.int32),
        ],
    )(energy.reshape(B, N))

    out = pl.pallas_call(
        _mask_body,
        grid=(B, n_chunks),
        in_specs=[
            pl.BlockSpec((1, C, rows, _LANES), lambda b, j: (b, 0, j, 0)),
            pl.BlockSpec((1, rows, _LANES), lambda b, j: (b, j, 0)),
            pl.BlockSpec((B, _L), lambda b, j: (0, 0)),
        ],
        out_specs=pl.BlockSpec((1, C, rows, _LANES), lambda b, j: (b, 0, j, 0)),
        out_shape=jax.ShapeDtypeStruct((B, C, rows_total, _LANES), jnp.float32),
    )(xf, energy, thresh)

    return out.reshape(B, C, H, W)


# speedup vs baseline: 1.0642x; 1.0036x over previous
import functools

import jax
import jax.numpy as jnp
from jax import lax
from jax.experimental import pallas as pl
from jax.experimental.pallas import tpu as pltpu
from jax.experimental.pallas import tpu_sc as plsc

_LANES = 128
_L = 16


def _energy_body(x_ref, e_ref):
    e_ref[...] = jnp.sum(x_ref[...], axis=1)


def _sc_threshold_body(N, k, n_shards, e_hbm, t_hbm, ebuf, kbuf, hist,
                       l256, tmp256, g256, tbuf, sh_hist):
    c = lax.axis_index("c")
    s = lax.axis_index("s")
    groups_per_core = 16 // n_shards
    b = c * groups_per_core + s // n_shards
    shard = s % n_shards
    shard_n = N // n_shards
    n_vecs = shard_n // _L

    lanes = lax.iota(jnp.int32, _L)
    ones = jnp.ones((_L,), jnp.int32)

    pltpu.sync_copy(e_hbm.at[b, pl.ds(shard * shard_n, shard_n)], ebuf)

    def prekey(i, _):
        for j in range(8):
            off = i * (8 * _L) + j * _L
            u = plsc.bitcast(ebuf[pl.ds(off, _L)], jnp.uint32)
            key = jnp.where(u >= jnp.uint32(0x80000000), ~u,
                            u | jnp.uint32(0x80000000))
            kbuf[pl.ds(off, _L)] = key
        return 0

    lax.fori_loop(0, n_vecs // 8, prekey, 0)

    t_part = jnp.uint32(0)
    r_k = jnp.int32(k)

    for p in range(4):
        shift = 24 - 8 * p

        def clr(i, _):
            for j in range(16):
                hist[i, pl.ds(j * _L, _L)] = jnp.zeros((_L,), jnp.int32)
            return 0

        lax.fori_loop(0, 16, clr, 0)

        def count(i, _):
            for j in range(8):
                off = i * (8 * _L) + j * _L
                key = kbuf[pl.ds(off, _L)]
                dig = (jnp.right_shift(key, jnp.uint32(shift))
                       & jnp.uint32(0xFF)).astype(jnp.int32)
                if p == 0:
                    plsc.addupdate_scatter(hist, [lanes, dig], ones)
                else:
                    m = jnp.right_shift(key, jnp.uint32(shift + 8)) == \
                        (t_part >> jnp.uint32(shift + 8))
                    plsc.addupdate_scatter(hist, [lanes, dig], ones, mask=m)
            return 0

        lax.fori_loop(0, n_vecs // 8, count, 0)

        def mrg(j, _):
            acc = hist[0, pl.ds(j * _L, _L)]
            for l in range(1, 16):
                acc = acc + hist[l, pl.ds(j * _L, _L)]
            l256[pl.ds(j * _L, _L)] = acc
            return 0

        lax.fori_loop(0, 16, mrg, 0)

        pltpu.sync_copy(l256, sh_hist.at[s])
        plsc.subcore_barrier()
        base_row = (s // n_shards) * n_shards
        for r in range(n_shards):
            pltpu.sync_copy(sh_hist.at[base_row + r], tmp256)

            def accj(j, _, first=(r == 0)):
                cur = tmp256[pl.ds(j * _L, _L)]
                if first:
                    g256[pl.ds(j * _L, _L)] = cur
                else:
                    g256[pl.ds(j * _L, _L)] = g256[pl.ds(j * _L, _L)] + cur
                return 0

            lax.fori_loop(0, 16, accj, 0)
        plsc.subcore_barrier()

        def scan_body(i, carry_in):
            carry, found, d, cgt = carry_in
            v = 15 - i
            hvec = g256[pl.ds(v * _L, _L)]
            ssum = jnp.sum(hvec)
            cum = plsc.cumsum(hvec)
            suff = ssum - cum + hvec
            cond = (carry + suff) >= r_k
            anyc = jnp.max(cond.astype(jnp.int32))
            jstar = jnp.max(jnp.where(cond, lanes, -1))
            sj = jnp.sum(jnp.where(lanes == jstar, suff, 0))
            hj = jnp.sum(jnp.where(lanes == jstar, hvec, 0))
            take = (found == 0) & (anyc == 1)
            d = jnp.where(take, v * _L + jstar, d)
            cgt = jnp.where(take, carry + sj - hj, cgt)
            found = jnp.where(take, 1, found)
            carry = carry + ssum
            return carry, found, d, cgt

        _, _, d, cgt = lax.fori_loop(
            0, 16, scan_body,
            (jnp.int32(0), jnp.int32(0), jnp.int32(0), jnp.int32(0)))
        r_k = r_k - cgt
        t_part = t_part | (d.astype(jnp.uint32) << jnp.uint32(shift))

    tvec = jnp.full((_L,), t_part, dtype=jnp.uint32)
    fvec = jnp.where(tvec >= jnp.uint32(0x80000000),
                     tvec ^ jnp.uint32(0x80000000), ~tvec)

    @pl.when(shard == 0)
    def _():
        tbuf[...] = plsc.bitcast(fvec, jnp.float32)
        pltpu.sync_copy(tbuf, t_hbm.at[b])


def _mask_body(x_ref, e_ref, t_ref, o_ref):
    b = pl.program_id(0)
    rows_b = jax.lax.broadcasted_iota(jnp.int32, t_ref.shape, 0)
    t = jnp.max(jnp.where(rows_b == b, t_ref[...], -jnp.inf))
    keep = e_ref[...] >= t
    o_ref[...] = jnp.where(keep[:, None, :, :], x_ref[...], jnp.float32(0.0))


@jax.jit
def kernel(x):
    B, C, H, W = x.shape
    N = H * W
    k = N // 2
    assert N % _LANES == 0
    rows_total = N // _LANES

    n_chunks = 4 if rows_total % 4 == 0 else 1
    rows = rows_total // n_chunks

    xf = x.reshape(B, C, rows_total, _LANES)

    energy = pl.pallas_call(
        _energy_body,
        grid=(B, n_chunks),
        in_specs=[pl.BlockSpec((1, C, rows, _LANES), lambda b, j: (b, 0, j, 0))],
        out_specs=pl.BlockSpec((1, rows, _LANES), lambda b, j: (b, j, 0)),
        out_shape=jax.ShapeDtypeStruct((B, rows_total, _LANES), jnp.float32),
    )(xf)

    assert 16 % (16 * B // 32) == 0 and B % 2 == 0
    n_shards = 32 // B
    shard_n = N // n_shards
    assert shard_n % (8 * _L) == 0 and (shard_n * n_shards) == N

    mesh = plsc.VectorSubcoreMesh(core_axis_name="c", subcore_axis_name="s")
    thresh = pl.kernel(
        functools.partial(_sc_threshold_body, N, k, n_shards),
        mesh=mesh,
        compiler_params=pltpu.CompilerParams(needs_layout_passes=False),
        out_type=jax.ShapeDtypeStruct((B, _L), jnp.float32),
        scratch_types=[
            pltpu.VMEM((shard_n,), jnp.float32),
            pltpu.VMEM((shard_n,), jnp.uint32),
            pltpu.VMEM((16, 256), jnp.int32),
            pltpu.VMEM((256,), jnp.int32),
            pltpu.VMEM((256,), jnp.int32),
            pltpu.VMEM((256,), jnp.int32),
            pltpu.VMEM((_L,), jnp.float32),
            pltpu.VMEM_SHARED((16, 256), jnp.int32),
        ],
    )(energy.reshape(B, N))

    out = pl.pallas_call(
        _mask_body,
        grid=(B, n_chunks),
        in_specs=[
            pl.BlockSpec((1, C, rows, _LANES), lambda b, j: (b, 0, j, 0)),
            pl.BlockSpec((1, rows, _LANES), lambda b, j: (b, j, 0)),
            pl.BlockSpec((B, _L), lambda b, j: (0, 0)),
        ],
        out_specs=pl.BlockSpec((1, C, rows, _LANES), lambda b, j: (b, 0, j, 0)),
        out_shape=jax.ShapeDtypeStruct((B, C, rows_total, _LANES), jnp.float32),
    )(xf, energy, thresh)

    return out.reshape(B, C, H, W)


# speedup vs baseline: 1.0864x; 1.0208x over previous
import functools

import jax
import jax.numpy as jnp
from jax import lax
from jax.experimental import pallas as pl
from jax.experimental.pallas import tpu as pltpu
from jax.experimental.pallas import tpu_sc as plsc

_LANES = 128
_L = 16


def _energy_body(x_ref, e_ref):
    e_ref[...] = jnp.sum(x_ref[...], axis=1)


def _sc_threshold_body(N, k, n_shards, e_hbm, t_hbm, ebuf, kbuf, cbuf, hist,
                       l256, tmp256, g256, tbuf, sh_hist):
    c = lax.axis_index("c")
    s = lax.axis_index("s")
    groups_per_core = 16 // n_shards
    b = c * groups_per_core + s // n_shards
    shard = s % n_shards
    shard_n = N // n_shards
    n_vecs = shard_n // _L

    lanes = lax.iota(jnp.int32, _L)
    ones = jnp.ones((_L,), jnp.int32)

    pltpu.sync_copy(e_hbm.at[b, pl.ds(shard * shard_n, shard_n)], ebuf)

    def clear_hist():
        def clr(i, _):
            for j in range(16):
                hist[i, pl.ds(j * _L, _L)] = jnp.zeros((_L,), jnp.int32)
            return 0

        lax.fori_loop(0, 16, clr, 0)

    def merge_and_pick(r_k):
        def mrg(j, _):
            acc = hist[0, pl.ds(j * _L, _L)]
            for l in range(1, 16):
                acc = acc + hist[l, pl.ds(j * _L, _L)]
            l256[pl.ds(j * _L, _L)] = acc
            return 0

        lax.fori_loop(0, 16, mrg, 0)

        pltpu.sync_copy(l256, sh_hist.at[s])
        plsc.subcore_barrier()
        base_row = (s // n_shards) * n_shards
        for r in range(n_shards):
            pltpu.sync_copy(sh_hist.at[base_row + r], tmp256)

            def accj(j, _, first=(r == 0)):
                cur = tmp256[pl.ds(j * _L, _L)]
                if first:
                    g256[pl.ds(j * _L, _L)] = cur
                else:
                    g256[pl.ds(j * _L, _L)] = g256[pl.ds(j * _L, _L)] + cur
                return 0

            lax.fori_loop(0, 16, accj, 0)
        plsc.subcore_barrier()

        def scan_body(i, carry_in):
            carry, found, d, cgt = carry_in
            v = 15 - i
            hvec = g256[pl.ds(v * _L, _L)]
            ssum = jnp.sum(hvec)
            cum = plsc.cumsum(hvec)
            suff = ssum - cum + hvec
            cond = (carry + suff) >= r_k
            anyc = jnp.max(cond.astype(jnp.int32))
            jstar = jnp.max(jnp.where(cond, lanes, -1))
            sj = jnp.sum(jnp.where(lanes == jstar, suff, 0))
            hj = jnp.sum(jnp.where(lanes == jstar, hvec, 0))
            take = (found == 0) & (anyc == 1)
            d = jnp.where(take, v * _L + jstar, d)
            cgt = jnp.where(take, carry + sj - hj, cgt)
            found = jnp.where(take, 1, found)
            carry = carry + ssum
            return carry, found, d, cgt

        _, _, d, cgt = lax.fori_loop(
            0, 16, scan_body,
            (jnp.int32(0), jnp.int32(0), jnp.int32(0), jnp.int32(0)))
        return d, cgt

    clear_hist()

    def p0(i, _):
        for j in range(8):
            off = i * (8 * _L) + j * _L
            u = plsc.bitcast(ebuf[pl.ds(off, _L)], jnp.uint32)
            key = jnp.where(u >= jnp.uint32(0x80000000), ~u,
                            u | jnp.uint32(0x80000000))
            kbuf[pl.ds(off, _L)] = key
            dig = jnp.right_shift(key, jnp.uint32(24)).astype(jnp.int32)
            plsc.addupdate_scatter(hist, [lanes, dig], ones)
        return 0

    lax.fori_loop(0, n_vecs // 8, p0, 0)
    d, cgt = merge_and_pick(jnp.int32(k))
    r_k = jnp.int32(k) - cgt
    t_part = d.astype(jnp.uint32) << jnp.uint32(24)

    clear_hist()
    pref8 = t_part >> jnp.uint32(24)

    def p1(i, off_c):
        for j in range(8):
            off = i * (8 * _L) + j * _L
            key = kbuf[pl.ds(off, _L)]
            m = (key >> jnp.uint32(24)) == pref8
            dig = ((key >> jnp.uint32(16)) & jnp.uint32(0xFF)).astype(jnp.int32)
            plsc.addupdate_scatter(hist, [lanes, dig], ones, mask=m)
            plsc.store_compressed(cbuf.at[pl.ds(off_c, _L)], key, mask=m)
            off_c = off_c + jnp.sum(m.astype(jnp.int32))
        return off_c

    m1 = lax.fori_loop(0, n_vecs // 8, p1, jnp.int32(0))
    d, cgt = merge_and_pick(r_k)
    r_k = r_k - cgt
    t_part = t_part | (d.astype(jnp.uint32) << jnp.uint32(16))

    clear_hist()
    pref16 = t_part >> jnp.uint32(16)

    def p2(i, off_c):
        base = i * _L
        key = cbuf[pl.ds(base, _L)]
        valid = (lanes + base) < m1
        m = ((key >> jnp.uint32(16)) == pref16) & valid
        dig = ((key >> jnp.uint32(8)) & jnp.uint32(0xFF)).astype(jnp.int32)
        plsc.addupdate_scatter(hist, [lanes, dig], ones, mask=m)
        plsc.store_compressed(kbuf.at[pl.ds(off_c, _L)], key, mask=m)
        return off_c + jnp.sum(m.astype(jnp.int32))

    g1 = (m1 + jnp.int32(_L - 1)) // jnp.int32(_L)
    m2 = lax.fori_loop(0, g1, p2, jnp.int32(0))
    d, cgt = merge_and_pick(r_k)
    r_k = r_k - cgt
    t_part = t_part | (d.astype(jnp.uint32) << jnp.uint32(8))

    clear_hist()
    pref24 = t_part >> jnp.uint32(8)

    def p3(i, _):
        base = i * _L
        key = kbuf[pl.ds(base, _L)]
        valid = (lanes + base) < m2
        m = ((key >> jnp.uint32(8)) == pref24) & valid
        dig = (key & jnp.uint32(0xFF)).astype(jnp.int32)
        plsc.addupdate_scatter(hist, [lanes, dig], ones, mask=m)
        return 0

    g2 = (m2 + jnp.int32(_L - 1)) // jnp.int32(_L)
    lax.fori_loop(0, g2, p3, 0)
    d, _ = merge_and_pick(r_k)
    t_part = t_part | d.astype(jnp.uint32)

    tvec = jnp.full((_L,), t_part, dtype=jnp.uint32)
    fvec = jnp.where(tvec >= jnp.uint32(0x80000000),
                     tvec ^ jnp.uint32(0x80000000), ~tvec)

    @pl.when(shard == 0)
    def _():
        tbuf[...] = plsc.bitcast(fvec, jnp.float32)
        pltpu.sync_copy(tbuf, t_hbm.at[b])


def _mask_body(x_ref, e_ref, t_ref, o_ref):
    b = pl.program_id(0)
    rows_b = jax.lax.broadcasted_iota(jnp.int32, t_ref.shape, 0)
    t = jnp.max(jnp.where(rows_b == b, t_ref[...], -jnp.inf))
    keep = e_ref[...] >= t
    o_ref[...] = jnp.where(keep[:, None, :, :], x_ref[...], jnp.float32(0.0))


@jax.jit
def kernel(x):
    B, C, H, W = x.shape
    N = H * W
    k = N // 2
    assert N % _LANES == 0
    rows_total = N // _LANES

    n_chunks = 4 if rows_total % 4 == 0 else 1
    rows = rows_total // n_chunks

    xf = x.reshape(B, C, rows_total, _LANES)

    energy = pl.pallas_call(
        _energy_body,
        grid=(B, n_chunks),
        in_specs=[pl.BlockSpec((1, C, rows, _LANES), lambda b, j: (b, 0, j, 0))],
        out_specs=pl.BlockSpec((1, rows, _LANES), lambda b, j: (b, j, 0)),
        out_shape=jax.ShapeDtypeStruct((B, rows_total, _LANES), jnp.float32),
    )(xf)

    assert 16 % (16 * B // 32) == 0 and B % 2 == 0
    n_shards = 32 // B
    shard_n = N // n_shards
    assert shard_n % (8 * _L) == 0 and (shard_n * n_shards) == N

    mesh = plsc.VectorSubcoreMesh(core_axis_name="c", subcore_axis_name="s")
    thresh = pl.kernel(
        functools.partial(_sc_threshold_body, N, k, n_shards),
        mesh=mesh,
        compiler_params=pltpu.CompilerParams(needs_layout_passes=False),
        out_type=jax.ShapeDtypeStruct((B, _L), jnp.float32),
        scratch_types=[
            pltpu.VMEM((shard_n,), jnp.float32),
            pltpu.VMEM((shard_n + _L,), jnp.uint32),
            pltpu.VMEM((shard_n + _L,), jnp.uint32),
            pltpu.VMEM((16, 256), jnp.int32),
            pltpu.VMEM((256,), jnp.int32),
            pltpu.VMEM((256,), jnp.int32),
            pltpu.VMEM((256,), jnp.int32),
            pltpu.VMEM((_L,), jnp.float32),
            pltpu.VMEM_SHARED((16, 256), jnp.int32),
        ],
    )(energy.reshape(B, N))

    out = pl.pallas_call(
        _mask_body,
        grid=(B, n_chunks),
        in_specs=[
            pl.BlockSpec((1, C, rows, _LANES), lambda b, j: (b, 0, j, 0)),
            pl.BlockSpec((1, rows, _LANES), lambda b, j: (b, j, 0)),
            pl.BlockSpec((B, _L), lambda b, j: (0, 0)),
        ],
        out_specs=pl.BlockSpec((1, C, rows, _LANES), lambda b, j: (b, 0, j, 0)),
        out_shape=jax.ShapeDtypeStruct((B, C, rows_total, _LANES), jnp.float32),
    )(xf, energy, thresh)

    return out.reshape(B, C, H, W)
